# baseline (speedup 1.0000x reference)
import jax
import jax.numpy as jnp
from jax.experimental import pallas as pl
from jax.experimental.pallas import tpu as pltpu

_N = 10000
_D = 128
_BM = 80


def _gcn_kernel(x_ref, a_ref, w_ref, o_ref):
    acc = jax.lax.dot_general(
        a_ref[...], x_ref[...],
        dimension_numbers=(((1,), (0,)), ((), ())),
        preferred_element_type=jnp.float32,
    )
    o_ref[...] = acc * w_ref[...]


def kernel(input, A, W):
    n, d = A.shape[0], input.shape[1]
    w2 = W.reshape(1, d)
    return pl.pallas_call(
        _gcn_kernel,
        grid=(n // _BM,),
        in_specs=[
            pl.BlockSpec((n, d), lambda i: (0, 0)),
            pl.BlockSpec((_BM, n), lambda i: (i, 0)),
            pl.BlockSpec((1, d), lambda i: (0, 0)),
        ],
        out_specs=pl.BlockSpec((_BM, d), lambda i: (i, 0)),
        out_shape=jax.ShapeDtypeStruct((n, d), jnp.float32),
        compiler_params=pltpu.CompilerParams(
            dimension_semantics=("parallel",),
        ),
    )(input, A, w2)


# speedup vs baseline: 1.3442x; 1.3442x over previous
import jax
import jax.numpy as jnp
from jax.experimental import pallas as pl
from jax.experimental.pallas import tpu as pltpu

_BM = 200
_NBUF = 4


def _gcn_kernel(x_hbm, a_hbm, w_ref, o_ref, xbuf, abuf, asem, xsem):
    n = a_hbm.shape[0]
    nstep = n // _BM

    pltpu.make_async_copy(x_hbm, xbuf, xsem).start()
    for j in range(_NBUF):
        pltpu.make_async_copy(
            a_hbm.at[pl.ds(j * _BM, _BM), :], abuf.at[j], asem.at[j]
        ).start()
    pltpu.make_async_copy(x_hbm, xbuf, xsem).wait()

    def loop(i, carry):
        slot = jax.lax.rem(i, _NBUF)
        pltpu.make_async_copy(
            a_hbm.at[pl.ds(i * _BM, _BM), :], abuf.at[slot], asem.at[slot]
        ).wait()
        acc = jax.lax.dot_general(
            abuf[slot], xbuf[...],
            dimension_numbers=(((1,), (0,)), ((), ())),
            preferred_element_type=jnp.float32,
        )
        o_ref[pl.ds(i * _BM, _BM), :] = acc * w_ref[...]

        @pl.when(i + _NBUF < nstep)
        def _():
            pltpu.make_async_copy(
                a_hbm.at[pl.ds((i + _NBUF) * _BM, _BM), :],
                abuf.at[slot], asem.at[slot],
            ).start()

        return carry

    jax.lax.fori_loop(0, nstep, loop, 0)


def kernel(input, A, W):
    n, d = A.shape[0], input.shape[1]
    w2 = W.reshape(1, d)
    return pl.pallas_call(
        _gcn_kernel,
        in_specs=[
            pl.BlockSpec(memory_space=pltpu.MemorySpace.HBM),
            pl.BlockSpec(memory_space=pltpu.MemorySpace.HBM),
            pl.BlockSpec(memory_space=pltpu.MemorySpace.VMEM),
        ],
        out_specs=pl.BlockSpec(memory_space=pltpu.MemorySpace.VMEM),
        out_shape=jax.ShapeDtypeStruct((n, d), jnp.float32),
        scratch_shapes=[
            pltpu.VMEM((n, d), jnp.float32),
            pltpu.VMEM((_NBUF, _BM, n), jnp.float32),
            pltpu.SemaphoreType.DMA((_NBUF,)),
            pltpu.SemaphoreType.DMA,
        ],
    )(input, A, w2)


# speedup vs baseline: 1.3728x; 1.0213x over previous
import jax
import jax.numpy as jnp
from jax.experimental import pallas as pl
from jax.experimental.pallas import tpu as pltpu

_N = 10000
_D = 128
_BM = 256


def _gcn_kernel(x_ref, a_ref, w_ref, o_ref):
    acc = jax.lax.dot_general(
        a_ref[...], x_ref[...],
        dimension_numbers=(((1,), (0,)), ((), ())),
        preferred_element_type=jnp.float32,
    )
    o_ref[...] = acc * w_ref[...]


def kernel(input, A, W):
    n, d = A.shape[0], input.shape[1]
    w2 = W.reshape(1, d)
    return pl.pallas_call(
        _gcn_kernel,
        grid=(pl.cdiv(n, _BM),),
        in_specs=[
            pl.BlockSpec((n, d), lambda i: (0, 0)),
            pl.BlockSpec((_BM, n), lambda i: (i, 0)),
            pl.BlockSpec((1, d), lambda i: (0, 0)),
        ],
        out_specs=pl.BlockSpec((_BM, d), lambda i: (i, 0)),
        out_shape=jax.ShapeDtypeStruct((n, d), jnp.float32),
        compiler_params=pltpu.CompilerParams(
            dimension_semantics=("parallel",),
        ),
    )(input, A, w2)
